# baseline (speedup 1.0000x reference)
import functools

import jax
import jax.numpy as jnp
from jax import lax
from jax.experimental import pallas as pl
from jax.experimental.pallas import tpu as pltpu
from jax.experimental.pallas import tpu_sc as plsc

NUM_CORES = 2
NUM_SUBCORES = 16
NW = NUM_CORES * NUM_SUBCORES
BBLK = 128
NBUF = 5
L = 16


@functools.partial(jax.jit, static_argnames=("hist", "emb_dim"))
def _emb_lookup(x_flat, table_t, hist, emb_dim):
    batch = x_flat.shape[0] // hist
    n_groups = hist // NBUF
    assert hist == n_groups * NBUF and batch == NW * BBLK

    mesh = plsc.VectorSubcoreMesh(core_axis_name="c", subcore_axis_name="s")

    @functools.partial(
        pl.kernel,
        out_type=jax.ShapeDtypeStruct((hist, emb_dim, batch), jnp.float32),
        mesh=mesh,
        scratch_types=[
            pltpu.VMEM((BBLK * hist,), jnp.int32),
            pltpu.VMEM((hist, BBLK), jnp.int32),
            pltpu.VMEM((NBUF, emb_dim, BBLK), jnp.float32),
            pltpu.SemaphoreType.DMA,
            [pltpu.SemaphoreType.DMA] * NBUF,
            [pltpu.SemaphoreType.DMA] * NBUF,
        ],
        compiler_params=pltpu.CompilerParams(
            use_tc_tiling_on_sc=False, needs_layout_passes=False
        ),
    )
    def emb_kernel(x_hbm, tab_hbm, out_hbm, idx_raw, idx_v, outt_v, isem,
                   gsems, wsems):
        c = lax.axis_index("c")
        s = lax.axis_index("s")
        wid = s * NUM_CORES + c
        b0 = wid * BBLK
        pltpu.async_copy(x_hbm.at[pl.ds(b0 * hist, BBLK * hist)], idx_raw,
                         isem).wait()

        lane = lax.iota(jnp.int32, L)
        lane_h = lane * hist

        def deinterleave(h, carry):
            for j in range(BBLK // L):
                src = lane_h + (j * L * hist) + h
                idx_v[h, pl.ds(j * L, L)] = plsc.load_gather(idx_raw, [src])
            return carry

        lax.fori_loop(0, hist, deinterleave, 0)

        for b in range(NBUF):
            pltpu.async_copy(
                outt_v.at[b], out_hbm.at[b, :, pl.ds(b0, BBLK)], wsems[b]
            )

        def body(k, carry):
            h0 = k * NBUF
            for b in range(NBUF):
                pltpu.make_async_copy(
                    out_hbm.at[b, :, pl.ds(b0, BBLK)], outt_v.at[b], wsems[b]
                ).wait()
                for e in range(emb_dim):
                    pltpu.async_copy(
                        tab_hbm.at[e].at[idx_v.at[h0 + b]],
                        outt_v.at[b].at[e],
                        gsems[b],
                    )
            for b in range(NBUF):
                pltpu.make_async_copy(
                    tab_hbm.at[:, pl.ds(0, BBLK)], outt_v.at[b], gsems[b]
                ).wait()
                pltpu.async_copy(
                    outt_v.at[b],
                    out_hbm.at[h0 + b, :, pl.ds(b0, BBLK)],
                    wsems[b],
                )
            return carry

        lax.fori_loop(0, n_groups, body, 0)
        for b in range(NBUF):
            pltpu.make_async_copy(
                out_hbm.at[b, :, pl.ds(b0, BBLK)], outt_v.at[b], wsems[b]
            ).wait()

    return emb_kernel(x_flat, table_t)


def kernel(x, table):
    batch, hist = x.shape
    vocab, emb_dim = table.shape
    assert batch == NW * BBLK
    x_flat = x.reshape(-1).astype(jnp.int32)
    out = _emb_lookup(x_flat, table.T, hist, emb_dim)
    return jnp.transpose(out, (2, 0, 1))


# speedup vs baseline: 4.1746x; 4.1746x over previous
import functools

import jax
import jax.numpy as jnp
from jax import lax
from jax.experimental import pallas as pl
from jax.experimental.pallas import tpu as pltpu
from jax.experimental.pallas import tpu_sc as plsc

NUM_CORES = 2
NUM_SUBCORES = 16
NW = NUM_CORES * NUM_SUBCORES
BBLK = 128
NBUF = 5
L = 16


@functools.partial(jax.jit, static_argnames=("hist", "emb_dim"))
def _emb_lookup(x_flat, table, hist, emb_dim):
    batch = x_flat.shape[0] // hist
    n_groups = hist // NBUF
    assert hist == n_groups * NBUF and batch == NW * BBLK

    mesh = plsc.VectorSubcoreMesh(core_axis_name="c", subcore_axis_name="s")

    @functools.partial(
        pl.kernel,
        out_type=jax.ShapeDtypeStruct((hist, emb_dim, batch), jnp.float32),
        mesh=mesh,
        scratch_types=[
            pltpu.VMEM((BBLK * hist,), jnp.int32),
            pltpu.VMEM((hist, BBLK), jnp.int32),
            pltpu.VMEM((NBUF, BBLK, emb_dim), jnp.float32),
            pltpu.VMEM((NBUF, emb_dim, BBLK), jnp.float32),
            pltpu.SemaphoreType.DMA,
            [pltpu.SemaphoreType.DMA] * NBUF,
            [pltpu.SemaphoreType.DMA] * NBUF,
        ],
        compiler_params=pltpu.CompilerParams(
            use_tc_tiling_on_sc=False, needs_layout_passes=False
        ),
    )
    def emb_kernel(x_hbm, tab_hbm, out_hbm, idx_raw, idx_v, rows_v, outt_v,
                   isem, gsems, wsems):
        c = lax.axis_index("c")
        s = lax.axis_index("s")
        wid = s * NUM_CORES + c
        b0 = wid * BBLK
        pltpu.async_copy(x_hbm.at[pl.ds(b0 * hist, BBLK * hist)], idx_raw,
                         isem).wait()

        lane = jax.lax.iota(jnp.int32, L)
        e_lo = lane
        e_hi = lane + L
        lane_h = lane * hist
        for h in range(hist):
            for j in range(BBLK // L):
                src = lane_h + (j * L * hist + h)
                v = plsc.load_gather(idx_raw, [src])
                idx_v[h, pl.ds(j * L, L)] = v

        def transpose_slot(b):
            for r in range(BBLK):
                col = jnp.full((L,), r, jnp.int32)
                v0 = rows_v[b, r, pl.ds(0, L)]
                v1 = rows_v[b, r, pl.ds(L, L)]
                plsc.store_scatter(outt_v.at[b], [e_lo, col], v0)
                plsc.store_scatter(outt_v.at[b], [e_hi, col], v1)

        def body(k, carry):
            h0 = k * NBUF
            for b in range(NBUF):
                pltpu.make_async_copy(
                    out_hbm.at[b, :, pl.ds(b0, BBLK)], outt_v.at[b], wsems[b]
                ).wait()
                pltpu.async_copy(
                    tab_hbm.at[idx_v.at[h0 + b]], rows_v.at[b], gsems[b]
                )
            for b in range(NBUF):
                pltpu.make_async_copy(
                    tab_hbm.at[idx_v.at[h0 + b]], rows_v.at[b], gsems[b]
                ).wait()
                transpose_slot(b)
                pltpu.async_copy(
                    outt_v.at[b],
                    out_hbm.at[h0 + b, :, pl.ds(b0, BBLK)],
                    wsems[b],
                )
            return carry

        for b in range(NBUF):
            pltpu.async_copy(
                outt_v.at[b], out_hbm.at[b, :, pl.ds(b0, BBLK)], wsems[b]
            )
        lax.fori_loop(0, n_groups, body, 0)
        for b in range(NBUF):
            pltpu.make_async_copy(
                out_hbm.at[b, :, pl.ds(b0, BBLK)], outt_v.at[b], wsems[b]
            ).wait()

    return emb_kernel(x_flat, table)


def kernel(x, table):
    batch, hist = x.shape
    vocab, emb_dim = table.shape
    assert emb_dim == 2 * L and batch == NW * BBLK
    x_flat = x.reshape(-1).astype(jnp.int32)
    out = _emb_lookup(x_flat, table, hist, emb_dim)
    return jnp.transpose(out, (2, 0, 1))


# speedup vs baseline: 4.1777x; 1.0007x over previous
import functools

import jax
import jax.numpy as jnp
from jax import lax
from jax.experimental import pallas as pl
from jax.experimental.pallas import tpu as pltpu
from jax.experimental.pallas import tpu_sc as plsc

NUM_CORES = 2
NUM_SUBCORES = 16
NW = NUM_CORES * NUM_SUBCORES
BBLK = 128
NBUF = 10
L = 16


@functools.partial(jax.jit, static_argnames=("hist", "emb_dim"))
def _emb_lookup(x_flat, table, hist, emb_dim):
    batch = x_flat.shape[0] // hist
    n_groups = hist // NBUF
    assert hist == n_groups * NBUF and batch == NW * BBLK

    mesh = plsc.VectorSubcoreMesh(core_axis_name="c", subcore_axis_name="s")

    @functools.partial(
        pl.kernel,
        out_type=jax.ShapeDtypeStruct((hist, emb_dim, batch), jnp.float32),
        mesh=mesh,
        scratch_types=[
            pltpu.VMEM((BBLK * hist,), jnp.int32),
            pltpu.VMEM((hist, BBLK), jnp.int32),
            pltpu.VMEM((NBUF, BBLK, emb_dim), jnp.float32),
            pltpu.VMEM((NBUF, emb_dim, BBLK), jnp.float32),
            pltpu.SemaphoreType.DMA,
            [pltpu.SemaphoreType.DMA] * NBUF,
            [pltpu.SemaphoreType.DMA] * NBUF,
        ],
        compiler_params=pltpu.CompilerParams(
            use_tc_tiling_on_sc=False, needs_layout_passes=False
        ),
    )
    def emb_kernel(x_hbm, tab_hbm, out_hbm, idx_raw, idx_v, rows_v, outt_v,
                   isem, gsems, wsems):
        c = lax.axis_index("c")
        s = lax.axis_index("s")
        wid = s * NUM_CORES + c
        b0 = wid * BBLK
        pltpu.async_copy(x_hbm.at[pl.ds(b0 * hist, BBLK * hist)], idx_raw,
                         isem).wait()

        lane = jax.lax.iota(jnp.int32, L)
        e_lo = lane
        e_hi = lane + L
        lane_h = lane * hist
        for h in range(hist):
            for j in range(BBLK // L):
                src = lane_h + (j * L * hist + h)
                v = plsc.load_gather(idx_raw, [src])
                idx_v[h, pl.ds(j * L, L)] = v

        def transpose_slot(b):
            for r in range(BBLK):
                col = jnp.full((L,), r, jnp.int32)
                v0 = rows_v[b, r, pl.ds(0, L)]
                v1 = rows_v[b, r, pl.ds(L, L)]
                plsc.store_scatter(outt_v.at[b], [e_lo, col], v0)
                plsc.store_scatter(outt_v.at[b], [e_hi, col], v1)

        def body(k, carry):
            h0 = k * NBUF
            for b in range(NBUF):
                pltpu.make_async_copy(
                    out_hbm.at[b, :, pl.ds(b0, BBLK)], outt_v.at[b], wsems[b]
                ).wait()
                pltpu.async_copy(
                    tab_hbm.at[idx_v.at[h0 + b]], rows_v.at[b], gsems[b]
                )
            for b in range(NBUF):
                pltpu.make_async_copy(
                    tab_hbm.at[idx_v.at[h0 + b]], rows_v.at[b], gsems[b]
                ).wait()
                transpose_slot(b)
                pltpu.async_copy(
                    outt_v.at[b],
                    out_hbm.at[h0 + b, :, pl.ds(b0, BBLK)],
                    wsems[b],
                )
            return carry

        for b in range(NBUF):
            pltpu.async_copy(
                outt_v.at[b], out_hbm.at[b, :, pl.ds(b0, BBLK)], wsems[b]
            )
        lax.fori_loop(0, n_groups, body, 0)
        for b in range(NBUF):
            pltpu.make_async_copy(
                out_hbm.at[b, :, pl.ds(b0, BBLK)], outt_v.at[b], wsems[b]
            ).wait()

    return emb_kernel(x_flat, table)


def kernel(x, table):
    batch, hist = x.shape
    vocab, emb_dim = table.shape
    assert emb_dim == 2 * L and batch == NW * BBLK
    x_flat = x.reshape(-1).astype(jnp.int32)
    out = _emb_lookup(x_flat, table, hist, emb_dim)
    return jnp.transpose(out, (2, 0, 1))
